# baseline (speedup 1.0000x reference)
from functools import partial, reduce

import numpy as np
import jax
import jax.numpy as jnp
from jax.experimental import pallas as pl
from jax.experimental.pallas import tpu as pltpu

_PASS_DIRS = (
    (1, 0, 0), (0, 1, 0),
    (1, 0, 0), (0, 0, 1),
    (0, 1, 0), (0, 0, 1),
    (1, 0, 0), (0, 1, 1),
    (1, 0, 0), (0, 1, -1),
    (0, 1, 0), (1, 0, 1),
    (0, 1, 0), (1, 0, -1),
    (0, 0, 1), (1, 1, 0),
    (0, 0, 1), (1, -1, 0),
)

_U32 = jnp.uint32


def _fill_slab(shape, fw):
    return jnp.full(shape, fw, _U32)


def _shift_h(x, dh, fw):
    if dh == 0:
        return x
    pad = _fill_slab(x.shape[:1] + (abs(dh),) + x.shape[2:], fw)
    if dh > 0:
        return jnp.concatenate([x[:, dh:], pad], axis=1)
    return jnp.concatenate([pad, x[:, :dh]], axis=1)


def _shift_w(x, dw, fw):
    if dw == 0:
        return x
    pad = _fill_slab(x.shape[:-1] + (abs(dw),), fw)
    if dw > 0:
        return jnp.concatenate([x[..., dw:], pad], axis=-1)
    return jnp.concatenate([pad, x[..., :dw]], axis=-1)


def _shift_d(x, dd, fw):
    if dd == 0:
        return x
    pad = _fill_slab((1,) + x.shape[1:], fw)
    if dd > 0:
        nxt = jnp.concatenate([x[1:], pad], axis=0)
        return (x >> 1) | (nxt << 31)
    prv = jnp.concatenate([pad, x[:-1]], axis=0)
    return (x << 1) | (prv >> 31)


def _tap(x, dd, dh, dw, fw):
    y = _shift_d(x, dd, fw)
    y = _shift_h(y, dh, fw)
    return _shift_w(y, dw, fw)


def _chain(x, fw):
    for dd, dh, dw in _PASS_DIRS:
        x = x | _tap(x, dd, dh, dw, fw) | _tap(x, -dd, -dh, -dw, fw)
    return x


def _fused_kernel(inp_ref, aux_ref, o_ref, vol_ref, *, nd):
    gi = pl.program_id(0) % 2
    k = pl.program_id(1)

    @pl.when(k < nd)
    def _pack():
        inp = inp_ref[0]
        aux = aux_ref[0]
        b = (aux < 0) | ((aux == 0) & (inp != 0))
        pw = _U32(1) << jax.lax.broadcasted_iota(_U32, (32, 1, 1), 0)
        bits = jnp.where(b, pw, _U32(0))
        while bits.shape[0] > 1:
            h = bits.shape[0] // 2
            bits = bits[:h] | bits[h:]
        vol_ref[nd - 1 - k] = bits[0] ^ (0 - gi).astype(_U32)

    @pl.when(k == nd - 1)
    def _morph():
        fw1 = (0 - gi).astype(_U32)
        fw2 = (gi - 1).astype(_U32)
        x = vol_ref[...]
        z = _chain(x, fw1)
        t = _chain(~z, fw2)
        vol_ref[...] = t ^ fw2

    @pl.when(k >= nd)
    def _unpack():
        one = jnp.float32(1)
        zero = jnp.float32(0)
        hh = o_ref.shape[2]
        h0 = pl.multiple_of((k - nd) * hh, hh)
        for p in range(nd):
            word = vol_ref[p, pl.ds(h0, hh)]
            o_ref[0, 32 * p:32 * (p + 1)] = jnp.stack(
                [jnp.where((word & _U32(1 << j)) != 0, one, zero)
                 for j in range(32)], axis=0)


def kernel(input, aux, device):
    del device
    B, C, D, H, W = input.shape
    ND = D // 32
    inp4 = input.reshape(B, D, H, W)
    aux4 = aux.reshape(B, D, H, W)

    def in_map(i, k):
        pack = k < ND
        return (jnp.where(pack, i, jnp.minimum(i + 1, B - 1)),
                jnp.where(pack, ND - 1 - k, ND - 1), 0, 0)
    out = pl.pallas_call(
        partial(_fused_kernel, nd=ND),
        grid=(B, ND + 2),
        in_specs=[
            pl.BlockSpec((1, 32, H, W), in_map),
            pl.BlockSpec((1, 32, H, W), in_map),
        ],
        out_specs=pl.BlockSpec(
            (1, D, H // 2, W),
            lambda i, k: (i, 0, jnp.maximum(k - ND, 0), 0)),
        out_shape=jax.ShapeDtypeStruct((B, D, H, W), jnp.float32),
        scratch_shapes=[pltpu.VMEM((ND, H, W), _U32)],
        compiler_params=pltpu.CompilerParams(
            dimension_semantics=("parallel", "arbitrary"),
            vmem_limit_bytes=60 * 1024 * 1024,
        ),
        name="morph_pool3d_fused",
    )(inp4, aux4)

    return out.reshape(B, C, D, H, W)


# speedup vs baseline: 1.0404x; 1.0404x over previous
from functools import partial, reduce

import numpy as np
import jax
import jax.numpy as jnp
from jax.experimental import pallas as pl
from jax.experimental.pallas import tpu as pltpu

_PASS_DIRS = (
    (1, 0, 0), (0, 1, 0),
    (1, 0, 0), (0, 0, 1),
    (0, 1, 0), (0, 0, 1),
    (1, 0, 0), (0, 1, 1),
    (1, 0, 0), (0, 1, -1),
    (0, 1, 0), (1, 0, 1),
    (0, 1, 0), (1, 0, -1),
    (0, 0, 1), (1, 1, 0),
    (0, 0, 1), (1, -1, 0),
)

_U32 = jnp.uint32


def _fill_slab(shape, fw):
    return jnp.full(shape, fw, _U32)


def _shift_h(x, dh, fw):
    if dh == 0:
        return x
    pad = _fill_slab(x.shape[:1] + (abs(dh),) + x.shape[2:], fw)
    if dh > 0:
        return jnp.concatenate([x[:, dh:], pad], axis=1)
    return jnp.concatenate([pad, x[:, :dh]], axis=1)


def _shift_w(x, dw, fw):
    if dw == 0:
        return x
    pad = _fill_slab(x.shape[:-1] + (abs(dw),), fw)
    if dw > 0:
        return jnp.concatenate([x[..., dw:], pad], axis=-1)
    return jnp.concatenate([pad, x[..., :dw]], axis=-1)


def _shift_d(x, dd, fw):
    if dd == 0:
        return x
    pad = _fill_slab((1,) + x.shape[1:], fw)
    if dd > 0:
        nxt = jnp.concatenate([x[1:], pad], axis=0)
        return (x >> 1) | (nxt << 31)
    prv = jnp.concatenate([pad, x[:-1]], axis=0)
    return (x << 1) | (prv >> 31)


def _tap(x, dd, dh, dw, fw):
    y = _shift_d(x, dd, fw)
    y = _shift_h(y, dh, fw)
    return _shift_w(y, dw, fw)


def _chain(x, fw):
    for dd, dh, dw in _PASS_DIRS:
        x = x | _tap(x, dd, dh, dw, fw) | _tap(x, -dd, -dh, -dw, fw)
    return x


def _fused_kernel(inp_ref, aux_ref, o_ref, vol_ref, *, nd):
    gi = pl.program_id(0) % 2
    k = pl.program_id(1)

    @pl.when(k < nd)
    def _pack():
        inp = inp_ref[0]
        aux = aux_ref[0]
        b = (aux < 0) | ((aux == 0) & (inp != 0))
        pw = _U32(1) << jax.lax.broadcasted_iota(_U32, (32, 1, 1), 0)
        bits = jnp.where(b, pw, _U32(0))
        while bits.shape[0] > 1:
            h = bits.shape[0] // 2
            bits = bits[:h] | bits[h:]
        vol_ref[nd - 1 - k] = bits[0] ^ (0 - gi).astype(_U32)

    @pl.when(k == nd - 1)
    def _morph():
        fw1 = (0 - gi).astype(_U32)
        fw2 = (gi - 1).astype(_U32)
        x = vol_ref[...]
        z = _chain(x, fw1)
        t = _chain(~z, fw2)
        vol_ref[...] = t ^ fw2

    @pl.when(k >= nd)
    def _unpack():
        word = vol_ref[k - nd]
        one = jnp.float32(1)
        zero = jnp.float32(0)
        o_ref[0] = jnp.stack(
            [jnp.where((word & _U32(1 << j)) != 0, one, zero)
             for j in range(32)], axis=0)


def kernel(input, aux, device):
    del device
    B, C, D, H, W = input.shape
    ND = D // 32
    inp4 = input.reshape(B, D, H, W)
    aux4 = aux.reshape(B, D, H, W)

    def in_map(i, k):
        pack = k < ND
        return (jnp.where(pack, i, jnp.minimum(i + 1, B - 1)),
                jnp.where(pack, ND - 1 - k, ND - 1), 0, 0)
    out = pl.pallas_call(
        partial(_fused_kernel, nd=ND),
        grid=(B, 2 * ND),
        in_specs=[
            pl.BlockSpec((1, 32, H, W), in_map),
            pl.BlockSpec((1, 32, H, W), in_map),
        ],
        out_specs=pl.BlockSpec(
            (1, 32, H, W), lambda i, k: (i, jnp.maximum(k - ND, 0), 0, 0)),
        out_shape=jax.ShapeDtypeStruct((B, D, H, W), jnp.float32),
        scratch_shapes=[pltpu.VMEM((ND, H, W), _U32)],
        compiler_params=pltpu.CompilerParams(
            dimension_semantics=("parallel", "arbitrary"),
            vmem_limit_bytes=48 * 1024 * 1024,
        ),
        name="morph_pool3d_fused",
    )(inp4, aux4)

    return out.reshape(B, C, D, H, W)
